# baseline (speedup 1.0000x reference)
import functools

import jax
import jax.numpy as jnp
from jax import lax
from jax.experimental import pallas as pl
from jax.experimental.pallas import tpu as pltpu
from jax.experimental.pallas import tpu_sc as plsc

_E = 16
_K = 2
_BLK = 1024

_NC = 2
_NS = 16
_NW = _NC * _NS


def _mm_block(x_ref, wt_ref, b_ref, o_ref):
    o_ref[...] = jnp.dot(x_ref[...], wt_ref[...],
                         preferred_element_type=jnp.float32) + b_ref[...]


def _tc_logits(x, wt, b2, T, H):
    return pl.pallas_call(
        _mm_block,
        grid=(T // _BLK,),
        in_specs=[
            pl.BlockSpec((_BLK, H), lambda i: (i, 0)),
            pl.BlockSpec((H, _E), lambda i: (0, 0)),
            pl.BlockSpec((1, _E), lambda i: (0, 0)),
        ],
        out_specs=pl.BlockSpec((_BLK, _E), lambda i: (i, 0)),
        out_shape=jax.ShapeDtypeStruct((T, _E), jnp.float32),
    )(x, wt, b2)


def _route_body(T, logits_hbm, probs_hbm, idx_hbm, map_hbm,
                ltile, pv, iv, mv):
    tpw = T // _NW
    wid = lax.axis_index("s") * _NC + lax.axis_index("c")
    base = wid * tpw
    pltpu.sync_copy(logits_hbm.at[pl.ds(base, tpw), :], ltile)
    iota = lax.iota(jnp.int32, 16)

    def _group(g, carry):
        pvec = jnp.zeros((16,), jnp.float32)
        ivec = jnp.zeros((16,), jnp.int32)
        for j in range(8):
            t = g * 8 + j
            v = ltile[t, :]
            m1 = jnp.max(v)
            i1 = plsc.all_reduce_ffs(v == m1)
            v2 = jnp.where(iota == i1, -3.0e38, v)
            m2 = jnp.max(v2)
            i2 = plsc.all_reduce_ffs(v2 == m2)
            s1 = 1.0 / (1.0 + jnp.exp(jnp.broadcast_to(-m1, (16,))))
            s2 = 1.0 / (1.0 + jnp.exp(jnp.broadcast_to(-m2, (16,))))
            ssum = s1 + s2
            w1 = s1 / ssum
            w2 = s2 / ssum
            mv[t, :] = jnp.where(iota == i1, w1,
                                 jnp.where(iota == i2, w2, 0.0))
            pvec = jnp.where(iota == 2 * j, w1,
                             jnp.where(iota == 2 * j + 1, w2, pvec))
            ivec = jnp.where(iota == 2 * j, i1,
                             jnp.where(iota == 2 * j + 1, i2, ivec))
        pv[pl.ds(g * 16, 16)] = pvec
        iv[pl.ds(g * 16, 16)] = ivec
        return carry

    lax.fori_loop(0, tpw // 8, _group, 0)
    pltpu.sync_copy(pv, probs_hbm.at[pl.ds(base * _K, tpw * _K)])
    pltpu.sync_copy(iv, idx_hbm.at[pl.ds(base * _K, tpw * _K)])
    pltpu.sync_copy(mv, map_hbm.at[pl.ds(base, tpw), :])


def _sc_route(logits_flat, T):
    tpw = T // _NW
    mesh = plsc.VectorSubcoreMesh(
        core_axis_name="c", subcore_axis_name="s",
        num_cores=_NC, num_subcores=_NS)
    k = pl.kernel(
        functools.partial(_route_body, T),
        mesh=mesh,
        out_type=[
            jax.ShapeDtypeStruct((T * _K,), jnp.float32),
            jax.ShapeDtypeStruct((T * _K,), jnp.int32),
            jax.ShapeDtypeStruct((T, _E), jnp.float32),
        ],
        scratch_types=[
            pltpu.VMEM((tpw, _E), jnp.float32),
            pltpu.VMEM((tpw * _K,), jnp.float32),
            pltpu.VMEM((tpw * _K,), jnp.int32),
            pltpu.VMEM((tpw, _E), jnp.float32),
        ],
        compiler_params=pltpu.CompilerParams(needs_layout_passes=False),
    )
    return k(logits_flat)


def kernel(hidden_states, W, b):
    B, S, H = hidden_states.shape
    T = B * S
    x = hidden_states.reshape(T, H)
    wt = W.T
    b2 = b.reshape(1, _E)
    logits = _tc_logits(x, wt, b2, T, H)
    probs_f, idx_f, rmap = _sc_route(logits, T)
    return (probs_f.reshape(B, S, _K), idx_f.reshape(B, S, _K),
            rmap.reshape(B, S, _E))


# speedup vs baseline: 2.8970x; 2.8970x over previous
import jax
import jax.numpy as jnp
from jax.experimental import pallas as pl
from jax.experimental.pallas import tpu as pltpu

_E = 16
_K = 2
_BLK = 1024


def _router_block(x_ref, w_ref, bt_ref, p_ref, i_ref, m_ref):
    lt = jax.lax.dot_general(
        w_ref[...], x_ref[...],
        dimension_numbers=(((1,), (1,)), ((), ())),
        preferred_element_type=jnp.float32) + bt_ref[...]
    p = jax.nn.sigmoid(lt)
    iota = jax.lax.broadcasted_iota(jnp.int32, p.shape, 0)
    m1 = jnp.max(p, axis=0, keepdims=True)
    i1 = jnp.min(jnp.where(p == m1, iota, _E), axis=0, keepdims=True)
    pm = jnp.where(iota == i1, -1.0, p)
    m2 = jnp.max(pm, axis=0, keepdims=True)
    i2 = jnp.min(jnp.where(pm == m2, iota, _E), axis=0, keepdims=True)
    s = m1 + m2
    w1 = m1 / s
    w2 = m2 / s
    p_ref[...] = jnp.concatenate([w1, w2], axis=0)
    i_ref[...] = jnp.concatenate([i1, i2], axis=0)
    m_ref[...] = jnp.where(iota == i1, w1, jnp.where(iota == i2, w2, 0.0))


def kernel(hidden_states, W, b):
    B, S, H = hidden_states.shape
    T = B * S
    x = hidden_states.reshape(T, H)
    bt = b.reshape(_E, 1)
    grid = (T // _BLK,)
    probs_t, idx_t, rmap_t = pl.pallas_call(
        _router_block,
        grid=grid,
        in_specs=[
            pl.BlockSpec((_BLK, H), lambda i: (i, 0)),
            pl.BlockSpec((_E, H), lambda i: (0, 0)),
            pl.BlockSpec((_E, 1), lambda i: (0, 0)),
        ],
        out_specs=[
            pl.BlockSpec((_K, _BLK), lambda i: (0, i)),
            pl.BlockSpec((_K, _BLK), lambda i: (0, i)),
            pl.BlockSpec((_E, _BLK), lambda i: (0, i)),
        ],
        out_shape=[
            jax.ShapeDtypeStruct((_K, T), jnp.float32),
            jax.ShapeDtypeStruct((_K, T), jnp.int32),
            jax.ShapeDtypeStruct((_E, T), jnp.float32),
        ],
        compiler_params=pltpu.CompilerParams(
            dimension_semantics=("parallel",)),
    )(x, W, bt)
    return (probs_t.T.reshape(B, S, _K), idx_t.T.reshape(B, S, _K),
            rmap_t.T.reshape(B, S, _E))
